# baseline (speedup 1.0000x reference)
import jax
import jax.numpy as jnp
from jax.experimental import pallas as pl
from jax.experimental.pallas import tpu as pltpu

TOP_K = 8
BLOCK_T = 1024

_MASKED = -2**31


def _router_body(x_ref, w_ref, vals_ref, idxs_ref):
    logits = jax.lax.dot_general(
        x_ref[...], w_ref[...],
        dimension_numbers=(((1,), (1,)), ((), ())),
        preferred_element_type=jnp.float32,
    )
    n_exp = logits.shape[1]
    bits = jax.lax.bitcast_convert_type(logits, jnp.int32)
    skey = jnp.where(bits >= 0, bits, bits ^ jnp.int32(0x7FFFFFFF))
    iota = jax.lax.broadcasted_iota(jnp.int32, logits.shape, 1)
    negidx = jnp.int32(n_exp - 1) - iota
    win_v, win_p = [], []
    for _ in range(TOP_K):
        w = jnp.max(skey, axis=1, keepdims=True)
        p = jnp.max(jnp.where(skey == w, negidx, jnp.int32(-1)),
                    axis=1, keepdims=True)
        win_v.append(w)
        win_p.append(p)
        skey = jnp.where(negidx == p, jnp.int32(_MASKED), skey)
    wv = jnp.concatenate(win_v, axis=1)
    wp = jnp.concatenate(win_p, axis=1)
    idxs_ref[...] = jnp.int32(n_exp - 1) - wp
    vb = jnp.where(wv >= 0, wv, wv ^ jnp.int32(0x7FFFFFFF))
    vals_ref[...] = jax.lax.bitcast_convert_type(vb, jnp.float32)


@jax.jit
def kernel(x, w_g):
    tokens, d = x.shape
    n_exp = w_g.shape[0]
    grid = (tokens // BLOCK_T,)
    vals, idxs = pl.pallas_call(
        _router_body,
        grid=grid,
        in_specs=[
            pl.BlockSpec((BLOCK_T, d), lambda i: (i, 0)),
            pl.BlockSpec((n_exp, d), lambda i: (0, 0)),
        ],
        out_specs=[
            pl.BlockSpec((BLOCK_T, TOP_K), lambda i: (i, 0)),
            pl.BlockSpec((BLOCK_T, TOP_K), lambda i: (i, 0)),
        ],
        out_shape=[
            jax.ShapeDtypeStruct((tokens, TOP_K), jnp.float32),
            jax.ShapeDtypeStruct((tokens, TOP_K), jnp.int32),
        ],
        compiler_params=pltpu.CompilerParams(
            dimension_semantics=("arbitrary",),
        ),
    )(x, w_g)
    return (vals, idxs)
